# Initial kernel scaffold; baseline (speedup 1.0000x reference)
import jax
import jax.numpy as jnp
from jax.experimental import pallas as pl


def kernel(h, pos, edge_index, W1, b1, W2, b2, W3, b3):
    raise NotImplementedError("write your pallas kernel here")


# speedup vs baseline: 2.6930x; 2.6930x over previous
import functools

import jax
import jax.numpy as jnp
from jax import lax
from jax.experimental import pallas as pl
from jax.experimental.pallas import tpu as pltpu
from jax.experimental.pallas import tpu_sc as plsc

NC = 2
NS = 16
L = 16


def _stage1_body(h_ref, w1a_ref, w1b_ref, b1_ref, p1_ref, p2_ref):
    h = h_ref[...]
    p1_ref[...] = jnp.dot(h, w1a_ref[...], preferred_element_type=jnp.float32)
    p2_ref[...] = (jnp.dot(h, w1b_ref[...], preferred_element_type=jnp.float32)
                   + b1_ref[...])


def _node_proj(h2, W1a, W1b, b1):
    N, D = h2.shape
    H = W1a.shape[1]
    BN = 1000
    return pl.pallas_call(
        _stage1_body,
        grid=(N // BN,),
        in_specs=[
            pl.BlockSpec((BN, D), lambda i: (i, 0)),
            pl.BlockSpec((D, H), lambda i: (0, 0)),
            pl.BlockSpec((D, H), lambda i: (0, 0)),
            pl.BlockSpec((1, H), lambda i: (0, 0)),
        ],
        out_specs=[
            pl.BlockSpec((BN, H), lambda i: (i, 0)),
            pl.BlockSpec((BN, H), lambda i: (i, 0)),
        ],
        out_shape=[jax.ShapeDtypeStruct((N, H), jnp.float32)] * 2,
    )(h2, W1a, W1b, b1)


def _edge_gather(P1, P2, src, dst, posx, posy, posz):
    N, H = P1.shape
    E = src.shape[0]
    NW = NC * NS
    EPW = E // NW
    C = 80
    NCH = EPW // C

    mesh = plsc.VectorSubcoreMesh(core_axis_name="c", subcore_axis_name="s")

    @functools.partial(
        pl.kernel,
        out_type=(jax.ShapeDtypeStruct((E, H), jnp.float32),
                  jax.ShapeDtypeStruct((E,), jnp.float32)),
        mesh=mesh,
        scratch_types=[
            pltpu.VMEM((EPW,), jnp.int32),
            pltpu.VMEM((EPW,), jnp.int32),
            pltpu.VMEM((N,), jnp.float32),
            pltpu.VMEM((N,), jnp.float32),
            pltpu.VMEM((N,), jnp.float32),
            pltpu.VMEM((C, H), jnp.float32),
            pltpu.VMEM((C, H), jnp.float32),
            pltpu.VMEM((C,), jnp.float32),
            pltpu.SemaphoreType.DMA,
            pltpu.SemaphoreType.DMA,
        ],
        compiler_params=pltpu.CompilerParams(needs_layout_passes=False),
    )
    def k(p1_hbm, p2_hbm, src_hbm, dst_hbm, px_hbm, py_hbm, pz_hbm,
          q_hbm, d2_hbm, srcv, dstv, px, py, pz, qb, qb2, d2b, sem, sem2):
        cid = lax.axis_index("c")
        sid = lax.axis_index("s")
        wid = sid * NC + cid
        base = wid * EPW
        pltpu.sync_copy(src_hbm.at[pl.ds(base, EPW)], srcv)
        pltpu.sync_copy(dst_hbm.at[pl.ds(base, EPW)], dstv)
        pltpu.sync_copy(px_hbm, px)
        pltpu.sync_copy(py_hbm, py)
        pltpu.sync_copy(pz_hbm, pz)

        @pl.loop(0, NCH)
        def _chunk(i):
            eb = i * C
            idx_s = srcv.at[pl.ds(eb, C)]
            idx_d = dstv.at[pl.ds(eb, C)]
            cp1 = pltpu.async_copy(p1_hbm.at[idx_s], qb, sem)
            cp2 = pltpu.async_copy(p2_hbm.at[idx_d], qb2, sem2)
            for j in range(C // L):
                s16 = srcv[pl.ds(eb + j * L, L)]
                d16 = dstv[pl.ds(eb + j * L, L)]
                dx = plsc.load_gather(px, [s16]) - plsc.load_gather(px, [d16])
                dy = plsc.load_gather(py, [s16]) - plsc.load_gather(py, [d16])
                dz = plsc.load_gather(pz, [s16]) - plsc.load_gather(pz, [d16])
                d2b[pl.ds(j * L, L)] = dx * dx + dy * dy + dz * dz
            cp1.wait()
            cp2.wait()

            @pl.loop(0, C)
            def _row(e):
                for j in range(H // L):
                    sl = pl.ds(j * L, L)
                    qb[e, sl] = qb[e, sl] + qb2[e, sl]

            pltpu.sync_copy(qb, q_hbm.at[pl.ds(base + eb, C)])
            pltpu.sync_copy(d2b, d2_hbm.at[pl.ds(base + eb, C)])

    return k(P1, P2, src, dst, posx, posy, posz)


def _stage3_body(q_ref, d2_ref, w1c_ref, w2_ref, b2_ref, out_ref):
    m = jnp.maximum(q_ref[...] + d2_ref[...] * w1c_ref[...], 0.0)
    r = jnp.dot(m, w2_ref[...], preferred_element_type=jnp.float32) + b2_ref[...]
    r = jnp.maximum(r, 0.0)
    Hh = out_ref.shape[-1]
    out_ref[0] = r[:, :Hh]
    out_ref[1] = r[:, Hh:]


def _edge_mlp(q, d2, w1c, W2, b2):
    E, H = q.shape
    BE = 512
    return pl.pallas_call(
        _stage3_body,
        grid=(E // BE,),
        in_specs=[
            pl.BlockSpec((BE, H), lambda i: (i, 0)),
            pl.BlockSpec((BE, 1), lambda i: (i, 0)),
            pl.BlockSpec((1, H), lambda i: (0, 0)),
            pl.BlockSpec((H, H), lambda i: (0, 0)),
            pl.BlockSpec((1, H), lambda i: (0, 0)),
        ],
        out_specs=pl.BlockSpec((2, BE, H // 2), lambda i: (0, i, 0)),
        out_shape=jax.ShapeDtypeStruct((2, E, H // 2), jnp.float32),
    )(q, d2, w1c, W2, b2)


def _scatter_add(m2, dst3, N):
    _, E, Hh = m2.shape
    EPS = E // NS
    C = 80
    NCH = EPS // C
    RPT = (N // NS) // 8 * 8
    RPT_LAST = N - (NS - 1) * RPT

    mesh = plsc.VectorSubcoreMesh(core_axis_name="c", subcore_axis_name="s")

    @functools.partial(
        pl.kernel,
        out_type=jax.ShapeDtypeStruct((2, N, Hh), jnp.float32),
        mesh=mesh,
        scratch_types=[
            pltpu.VMEM((NCH, C), jnp.int32),
            pltpu.VMEM((C, Hh), jnp.float32),
            pltpu.VMEM((L, Hh), jnp.float32),
            pltpu.VMEM_SHARED((N, Hh), jnp.float32),
            pltpu.SemaphoreType.DMA,
        ],
    )
    def k(m_hbm, dst_hbm, out_hbm, idxb, mbuf, zbuf, acc, sem):
        cid = lax.axis_index("c")
        sid = lax.axis_index("s")
        ebase = sid * EPS
        pltpu.sync_copy(dst_hbm.at[sid], idxb)
        zero = jnp.zeros((L,), jnp.float32)

        @pl.loop(0, L)
        def _z(r):
            for j in range(Hh // L):
                zbuf[r, pl.ds(j * L, L)] = zero

        rbase = sid * RPT

        @pl.when(sid < NS - 1)
        def _():
            @pl.loop(0, RPT // L)
            def _zc(t):
                pltpu.sync_copy(zbuf, acc.at[pl.ds(rbase + t * L, L)])

        @pl.when(sid == NS - 1)
        def _():
            @pl.loop(0, RPT_LAST // L)
            def _zc(t):
                pltpu.sync_copy(zbuf, acc.at[pl.ds(rbase + t * L, L)])

        plsc.subcore_barrier()

        @pl.loop(0, NCH)
        def _chunk(i):
            pltpu.sync_copy(m_hbm.at[cid, pl.ds(ebase + i * C, C)], mbuf)
            pltpu.sync_copy(mbuf, acc.at[idxb.at[i]], add=True)

        plsc.subcore_barrier()

        @pl.when(sid < NS - 1)
        def _():
            pltpu.sync_copy(acc.at[pl.ds(rbase, RPT)],
                            out_hbm.at[cid, pl.ds(rbase, RPT)])

        @pl.when(sid == NS - 1)
        def _():
            pltpu.sync_copy(acc.at[pl.ds(rbase, RPT_LAST)],
                            out_hbm.at[cid, pl.ds(rbase, RPT_LAST)])

    return k(m2, dst3)


def _stage5_body(magg_ref, h_ref, w3_ref, b3_ref, out_ref):
    r = (jnp.dot(magg_ref[0], w3_ref[0], preferred_element_type=jnp.float32)
         + jnp.dot(magg_ref[1], w3_ref[1], preferred_element_type=jnp.float32)
         + b3_ref[...])
    out_ref[...] = h_ref[...] + jnp.maximum(r, 0.0)


def _node_update(magg, h2, W3s, b3):
    N, D = h2.shape
    Hh = W3s.shape[1]
    BN = 1000
    return pl.pallas_call(
        _stage5_body,
        grid=(N // BN,),
        in_specs=[
            pl.BlockSpec((2, BN, Hh), lambda i: (0, i, 0)),
            pl.BlockSpec((BN, D), lambda i: (i, 0)),
            pl.BlockSpec((2, Hh, D), lambda i: (0, 0, 0)),
            pl.BlockSpec((1, D), lambda i: (0, 0)),
        ],
        out_specs=pl.BlockSpec((BN, D), lambda i: (i, 0)),
        out_shape=jax.ShapeDtypeStruct((N, D), jnp.float32),
    )(magg, h2, W3s, b3)


def kernel(h, pos, edge_index, W1, b1, W2, b2, W3, b3):
    B, N, D = h.shape
    E = edge_index.shape[1]
    H = W2.shape[0]

    h2 = h[0]
    posx, posy, posz = pos[0, :, 0], pos[0, :, 1], pos[0, :, 2]
    src = edge_index[0]
    dst = edge_index[1]
    W1a = W1[:D]
    W1b = W1[D:2 * D]
    w1c = W1[2 * D:2 * D + 1]

    P1, P2 = _node_proj(h2, W1a, W1b, b1.reshape(1, H))
    q, d2 = _edge_gather(P1, P2, src, dst, posx, posy, posz)
    m2 = _edge_mlp(q, d2.reshape(E, 1), w1c, W2, b2.reshape(1, H))
    dst3 = dst.reshape(NS, (E // NS) // 80, 80)
    magg = _scatter_add(m2, dst3, N)
    h_out = _node_update(magg, h2, W3.reshape(2, H // 2, D), b3.reshape(1, D))
    return (h_out[None], pos)


# speedup vs baseline: 3.8783x; 1.4401x over previous
import functools

import jax
import jax.numpy as jnp
from jax import lax
from jax.experimental import pallas as pl
from jax.experimental.pallas import tpu as pltpu
from jax.experimental.pallas import tpu_sc as plsc

NC = 2
NS = 16
L = 16


def _pack_bf16_pair(a, b):
    au = lax.bitcast_convert_type(a, jnp.uint32)
    bu = lax.bitcast_convert_type(b, jnp.uint32)
    au = (au + jnp.uint32(0x7FFF) + ((au >> 16) & jnp.uint32(1))) >> 16
    bu = (bu + jnp.uint32(0x7FFF) + ((bu >> 16) & jnp.uint32(1))) \
        & jnp.uint32(0xFFFF0000)
    return lax.bitcast_convert_type(au | bu, jnp.int32)


def _unpack_bf16_pair(p):
    pu = lax.bitcast_convert_type(p, jnp.uint32)
    lo = lax.bitcast_convert_type(pu << 16, jnp.float32)
    hi = lax.bitcast_convert_type(pu & jnp.uint32(0xFFFF0000), jnp.float32)
    return jnp.concatenate([lo, hi], axis=1)


def _stage1_body(h_ref, w1a_ref, w1b_ref, b1_ref, p1_ref, p2_ref):
    h = h_ref[...]
    r1 = jnp.dot(h, w1a_ref[...], preferred_element_type=jnp.float32)
    r2 = (jnp.dot(h, w1b_ref[...], preferred_element_type=jnp.float32)
          + b1_ref[...])
    Hh = r1.shape[1] // 2
    p1_ref[...] = _pack_bf16_pair(r1[:, :Hh], r1[:, Hh:])
    p2_ref[...] = _pack_bf16_pair(r2[:, :Hh], r2[:, Hh:])


def _node_proj(h2, W1a, W1b, b1):
    N, D = h2.shape
    H = W1a.shape[1]
    BN = 1000
    return pl.pallas_call(
        _stage1_body,
        grid=(N // BN,),
        in_specs=[
            pl.BlockSpec((BN, D), lambda i: (i, 0)),
            pl.BlockSpec((D, H), lambda i: (0, 0)),
            pl.BlockSpec((D, H), lambda i: (0, 0)),
            pl.BlockSpec((1, H), lambda i: (0, 0)),
        ],
        out_specs=[
            pl.BlockSpec((BN, H // 2), lambda i: (i, 0)),
            pl.BlockSpec((BN, H // 2), lambda i: (i, 0)),
        ],
        out_shape=[jax.ShapeDtypeStruct((N, H // 2), jnp.int32)] * 2,
    )(h2, W1a, W1b, b1)


def _edge_gather(P1, P2, src, dst, posx, posy, posz):
    N, Hw = P1.shape
    E = src.shape[0]
    NW = NC * NS
    EPW = E // NW
    C = 80
    NCH = EPW // C

    mesh = plsc.VectorSubcoreMesh(core_axis_name="c", subcore_axis_name="s")

    @functools.partial(
        pl.kernel,
        out_type=(jax.ShapeDtypeStruct((E, Hw), jnp.int32),
                  jax.ShapeDtypeStruct((E,), jnp.float32)),
        mesh=mesh,
        scratch_types=[
            pltpu.VMEM((EPW,), jnp.int32),
            pltpu.VMEM((EPW,), jnp.int32),
            pltpu.VMEM((N,), jnp.float32),
            pltpu.VMEM((N,), jnp.float32),
            pltpu.VMEM((N,), jnp.float32),
            pltpu.VMEM((2, C, Hw), jnp.int32),
            pltpu.VMEM((2, C, Hw), jnp.int32),
            pltpu.VMEM((2, C), jnp.float32),
            [pltpu.SemaphoreType.DMA] * 2,
            [pltpu.SemaphoreType.DMA] * 2,
            [pltpu.SemaphoreType.DMA] * 2,
        ],
        compiler_params=pltpu.CompilerParams(needs_layout_passes=False),
    )
    def k(p1_hbm, p2_hbm, src_hbm, dst_hbm, px_hbm, py_hbm, pz_hbm,
          q_hbm, d2_hbm, srcv, dstv, px, py, pz, g1, g2, d2b,
          semg1, semg2, semw):
        cid = lax.axis_index("c")
        sid = lax.axis_index("s")
        wid = sid * NC + cid
        base = wid * EPW
        pltpu.sync_copy(src_hbm.at[pl.ds(base, EPW)], srcv)
        pltpu.sync_copy(dst_hbm.at[pl.ds(base, EPW)], dstv)
        pltpu.sync_copy(px_hbm, px)
        pltpu.sync_copy(py_hbm, py)
        pltpu.sync_copy(pz_hbm, pz)

        def gather_copies(c, s):
            eb = c * C
            cp1 = pltpu.make_async_copy(p1_hbm.at[srcv.at[pl.ds(eb, C)]],
                                        g1.at[s], semg1[s])
            cp2 = pltpu.make_async_copy(p2_hbm.at[dstv.at[pl.ds(eb, C)]],
                                        g2.at[s], semg2[s])
            return cp1, cp2

        def writeout_copies(c, s):
            eb = c * C
            cq = pltpu.make_async_copy(g1.at[s], q_hbm.at[pl.ds(base + eb, C)],
                                       semw[s])
            cd = pltpu.make_async_copy(d2b.at[s], d2_hbm.at[pl.ds(base + eb, C)],
                                       semw[s])
            return cq, cd

        def issue(copies):
            for cp in copies:
                cp.start()

        def wait(copies):
            for cp in copies:
                cp.wait()

        def body(c, s, tail=False):
            wait(gather_copies(c, s))
            if tail:
                wait(writeout_copies(c - 1, s ^ 1))
            else:
                @pl.when(c >= 1)
                def _():
                    wait(writeout_copies(c - 1, s ^ 1))

                issue(gather_copies(c + 1, s ^ 1))

            eb = c * C
            for j in range(C // L):
                s16 = srcv[pl.ds(eb + j * L, L)]
                d16 = dstv[pl.ds(eb + j * L, L)]
                dx = plsc.load_gather(px, [s16]) - plsc.load_gather(px, [d16])
                dy = plsc.load_gather(py, [s16]) - plsc.load_gather(py, [d16])
                dz = plsc.load_gather(pz, [s16]) - plsc.load_gather(pz, [d16])
                d2b[s, pl.ds(j * L, L)] = dx * dx + dy * dy + dz * dz

            @pl.loop(0, C)
            def _row(e):
                for j in range(Hw // L):
                    sl = pl.ds(j * L, L)
                    a = plsc.bitcast(g1[s, e, sl], jnp.bfloat16)
                    b = plsc.bitcast(g2[s, e, sl], jnp.bfloat16)
                    g1[s, e, sl] = plsc.bitcast(a + b, jnp.int32)

            issue(writeout_copies(c, s))

        issue(gather_copies(0, 0))
        npair = (NCH - 1) // 2

        @pl.loop(0, npair)
        def _pair(t):
            body(2 * t, 0)
            body(2 * t + 1, 1)

        body(NCH - 1, 0, tail=True)
        wait(writeout_copies(NCH - 1, 0))

    return k(P1, P2, src, dst, posx, posy, posz)


def _stage3_body(q_ref, d2_ref, w1c_ref, w2_ref, b2_ref, out_ref):
    q = _unpack_bf16_pair(q_ref[...])
    m = jnp.maximum(q + d2_ref[...] * w1c_ref[...], 0.0)
    r = jnp.dot(m.astype(jnp.bfloat16), w2_ref[...],
                preferred_element_type=jnp.float32) + b2_ref[...]
    r = jnp.maximum(r, 0.0)
    Hh = out_ref.shape[-1]
    out_ref[0] = r[:, :Hh]
    out_ref[1] = r[:, Hh:]


def _edge_mlp(q, d2, w1c, W2, b2):
    E, Hw = q.shape
    H = 2 * Hw
    BE = 512
    return pl.pallas_call(
        _stage3_body,
        grid=(E // BE,),
        in_specs=[
            pl.BlockSpec((BE, Hw), lambda i: (i, 0)),
            pl.BlockSpec((BE, 1), lambda i: (i, 0)),
            pl.BlockSpec((1, H), lambda i: (0, 0)),
            pl.BlockSpec((H, H), lambda i: (0, 0)),
            pl.BlockSpec((1, H), lambda i: (0, 0)),
        ],
        out_specs=pl.BlockSpec((2, BE, H // 2), lambda i: (0, i, 0)),
        out_shape=jax.ShapeDtypeStruct((2, E, H // 2), jnp.float32),
    )(q, d2, w1c, W2, b2)


def _scatter_add(m2, dst3, N):
    _, E, Hh = m2.shape
    EPS = E // NS
    C = 80
    NCH = EPS // C
    RPT = (N // NS) // 8 * 8
    RPT_LAST = N - (NS - 1) * RPT

    mesh = plsc.VectorSubcoreMesh(core_axis_name="c", subcore_axis_name="s")

    @functools.partial(
        pl.kernel,
        out_type=jax.ShapeDtypeStruct((2, N, Hh), jnp.float32),
        mesh=mesh,
        scratch_types=[
            pltpu.VMEM((128, C), jnp.int32),
            pltpu.VMEM((2, C, Hh), jnp.float32),
            pltpu.VMEM((L, Hh), jnp.float32),
            pltpu.VMEM_SHARED((N, Hh), jnp.float32),
            [pltpu.SemaphoreType.DMA] * 2,
        ],
    )
    def k(m_hbm, dst_hbm, out_hbm, idxb, mbuf, zbuf, acc, seml):
        cid = lax.axis_index("c")
        sid = lax.axis_index("s")
        ebase = sid * EPS
        zero = jnp.zeros((L,), jnp.float32)

        @pl.loop(0, L)
        def _z(r):
            for j in range(Hh // L):
                zbuf[r, pl.ds(j * L, L)] = zero

        rbase = sid * RPT

        @pl.when(sid < NS - 1)
        def _():
            @pl.loop(0, RPT // L)
            def _zc(t):
                pltpu.sync_copy(zbuf, acc.at[pl.ds(rbase + t * L, L)])

        @pl.when(sid == NS - 1)
        def _():
            @pl.loop(0, RPT_LAST // L)
            def _zc(t):
                pltpu.sync_copy(zbuf, acc.at[pl.ds(rbase + t * L, L)])

        plsc.subcore_barrier()

        def load_copy(c, s):
            return pltpu.make_async_copy(
                m_hbm.at[cid, pl.ds(ebase + c * C, C)], mbuf.at[s], seml[s])

        def phase(cbase, nph):
            pltpu.sync_copy(dst_hbm.at[sid, pl.ds(cbase, nph)],
                            idxb.at[pl.ds(0, nph)])
            load_copy(cbase, 0).start()

            @pl.loop(0, nph // 2)
            def _chunk(t):
                for s in (0, 1):
                    k = 2 * t + s
                    c = cbase + k
                    load_copy(c, s).wait()

                    @pl.when(k + 1 < nph)
                    def _():
                        load_copy(c + 1, s ^ 1).start()

                    pltpu.sync_copy(mbuf.at[s], acc.at[idxb.at[k]], add=True)

        phase(0, 128)
        phase(128, NCH - 128)

        plsc.subcore_barrier()

        @pl.when(sid < NS - 1)
        def _():
            pltpu.sync_copy(acc.at[pl.ds(rbase, RPT)],
                            out_hbm.at[cid, pl.ds(rbase, RPT)])

        @pl.when(sid == NS - 1)
        def _():
            pltpu.sync_copy(acc.at[pl.ds(rbase, RPT_LAST)],
                            out_hbm.at[cid, pl.ds(rbase, RPT_LAST)])

    return k(m2, dst3)


def _stage5_body(magg_ref, h_ref, w3_ref, b3_ref, out_ref):
    r = (jnp.dot(magg_ref[0], w3_ref[0], preferred_element_type=jnp.float32)
         + jnp.dot(magg_ref[1], w3_ref[1], preferred_element_type=jnp.float32)
         + b3_ref[...])
    out_ref[...] = h_ref[...] + jnp.maximum(r, 0.0)


def _node_update(magg, h2, W3s, b3):
    N, D = h2.shape
    Hh = W3s.shape[1]
    BN = 1000
    return pl.pallas_call(
        _stage5_body,
        grid=(N // BN,),
        in_specs=[
            pl.BlockSpec((2, BN, Hh), lambda i: (0, i, 0)),
            pl.BlockSpec((BN, D), lambda i: (i, 0)),
            pl.BlockSpec((2, Hh, D), lambda i: (0, 0, 0)),
            pl.BlockSpec((1, D), lambda i: (0, 0)),
        ],
        out_specs=pl.BlockSpec((BN, D), lambda i: (i, 0)),
        out_shape=jax.ShapeDtypeStruct((N, D), jnp.float32),
    )(magg, h2, W3s, b3)


def kernel(h, pos, edge_index, W1, b1, W2, b2, W3, b3):
    B, N, D = h.shape
    E = edge_index.shape[1]
    H = W2.shape[0]

    h2 = h[0]
    posx, posy, posz = pos[0, :, 0], pos[0, :, 1], pos[0, :, 2]
    src = edge_index[0]
    dst = edge_index[1]
    W1a = W1[:D]
    W1b = W1[D:2 * D]
    w1c = W1[2 * D:2 * D + 1]

    P1, P2 = _node_proj(h2, W1a, W1b, b1.reshape(1, H))
    q, d2 = _edge_gather(P1, P2, src, dst, posx, posy, posz)
    m2 = _edge_mlp(q, d2.reshape(E, 1), w1c, W2.astype(jnp.bfloat16),
                   b2.reshape(1, H))
    dst3 = dst.reshape(NS, (E // NS) // 80, 80)
    magg = _scatter_add(m2, dst3, N)
    h_out = _node_update(magg, h2, W3.reshape(2, H // 2, D), b3.reshape(1, D))
    return (h_out[None], pos)


# speedup vs baseline: 4.4569x; 1.1492x over previous
import functools

import jax
import jax.numpy as jnp
from jax import lax
from jax.experimental import pallas as pl
from jax.experimental.pallas import tpu as pltpu
from jax.experimental.pallas import tpu_sc as plsc

NC = 2
NS = 16
L = 16


def _pack_bf16_pair(a, b):
    au = lax.bitcast_convert_type(a, jnp.uint32)
    bu = lax.bitcast_convert_type(b, jnp.uint32)
    au = (au + jnp.uint32(0x7FFF) + ((au >> 16) & jnp.uint32(1))) >> 16
    bu = (bu + jnp.uint32(0x7FFF) + ((bu >> 16) & jnp.uint32(1))) \
        & jnp.uint32(0xFFFF0000)
    return lax.bitcast_convert_type(au | bu, jnp.int32)


def _unpack_bf16_halves(p):
    pu = lax.bitcast_convert_type(p, jnp.uint32)
    lo = lax.bitcast_convert_type(pu << 16, jnp.float32)
    hi = lax.bitcast_convert_type(pu & jnp.uint32(0xFFFF0000), jnp.float32)
    return lo, hi


def _stage1_body(h_ref, w1a_ref, w1b_ref, b1_ref, p1_ref, p2_ref):
    h = h_ref[...]
    r1 = jnp.dot(h, w1a_ref[...], preferred_element_type=jnp.float32)
    r2 = (jnp.dot(h, w1b_ref[...], preferred_element_type=jnp.float32)
          + b1_ref[...])
    Hh = r1.shape[1] // 2
    p1_ref[...] = _pack_bf16_pair(r1[:, :Hh], r1[:, Hh:])
    p2_ref[...] = _pack_bf16_pair(r2[:, :Hh], r2[:, Hh:])


def _node_proj(h2, W1a, W1b, b1):
    N, D = h2.shape
    H = W1a.shape[1]
    BN = 1000
    return pl.pallas_call(
        _stage1_body,
        grid=(N // BN,),
        in_specs=[
            pl.BlockSpec((BN, D), lambda i: (i, 0)),
            pl.BlockSpec((D, H), lambda i: (0, 0)),
            pl.BlockSpec((D, H), lambda i: (0, 0)),
            pl.BlockSpec((1, H), lambda i: (0, 0)),
        ],
        out_specs=[
            pl.BlockSpec((BN, H // 2), lambda i: (i, 0)),
            pl.BlockSpec((BN, H // 2), lambda i: (i, 0)),
        ],
        out_shape=[jax.ShapeDtypeStruct((N, H // 2), jnp.int32)] * 2,
    )(h2, W1a, W1b, b1)


def _edge_gather(P1, P2, src, dst, posx, posy, posz):
    N, Hw = P1.shape
    E = src.shape[0]
    NW = NC * NS
    EPW = E // NW
    C = 80
    NCH = EPW // C

    mesh = plsc.VectorSubcoreMesh(core_axis_name="c", subcore_axis_name="s")

    @functools.partial(
        pl.kernel,
        out_type=(jax.ShapeDtypeStruct((E, Hw), jnp.int32),
                  jax.ShapeDtypeStruct((E,), jnp.float32)),
        mesh=mesh,
        scratch_types=[
            pltpu.VMEM((EPW,), jnp.int32),
            pltpu.VMEM((EPW,), jnp.int32),
            pltpu.VMEM((N,), jnp.float32),
            pltpu.VMEM((N,), jnp.float32),
            pltpu.VMEM((N,), jnp.float32),
            pltpu.VMEM((2, C, Hw), jnp.int32),
            pltpu.VMEM((2, C, Hw), jnp.int32),
            pltpu.VMEM((2, C), jnp.float32),
            [pltpu.SemaphoreType.DMA] * 2,
            [pltpu.SemaphoreType.DMA] * 2,
            [pltpu.SemaphoreType.DMA] * 2,
        ],
        compiler_params=pltpu.CompilerParams(needs_layout_passes=False),
    )
    def k(p1_hbm, p2_hbm, src_hbm, dst_hbm, px_hbm, py_hbm, pz_hbm,
          q_hbm, d2_hbm, srcv, dstv, px, py, pz, g1, g2, d2b,
          semg1, semg2, semw):
        cid = lax.axis_index("c")
        sid = lax.axis_index("s")
        wid = sid * NC + cid
        base = wid * EPW
        pltpu.sync_copy(src_hbm.at[pl.ds(base, EPW)], srcv)
        pltpu.sync_copy(dst_hbm.at[pl.ds(base, EPW)], dstv)
        pltpu.sync_copy(px_hbm, px)
        pltpu.sync_copy(py_hbm, py)
        pltpu.sync_copy(pz_hbm, pz)

        def gather_copies(c, s):
            eb = c * C
            cp1 = pltpu.make_async_copy(p1_hbm.at[srcv.at[pl.ds(eb, C)]],
                                        g1.at[s], semg1[s])
            cp2 = pltpu.make_async_copy(p2_hbm.at[dstv.at[pl.ds(eb, C)]],
                                        g2.at[s], semg2[s])
            return cp1, cp2

        def writeout_copies(c, s):
            eb = c * C
            cq = pltpu.make_async_copy(g1.at[s], q_hbm.at[pl.ds(base + eb, C)],
                                       semw[s])
            cd = pltpu.make_async_copy(d2b.at[s], d2_hbm.at[pl.ds(base + eb, C)],
                                       semw[s])
            return cq, cd

        def issue(copies):
            for cp in copies:
                cp.start()

        def wait(copies):
            for cp in copies:
                cp.wait()

        def body(c, s, tail=False):
            wait(gather_copies(c, s))
            if tail:
                wait(writeout_copies(c - 1, s ^ 1))
            else:
                @pl.when(c >= 1)
                def _():
                    wait(writeout_copies(c - 1, s ^ 1))

                issue(gather_copies(c + 1, s ^ 1))

            eb = c * C
            for j in range(C // L):
                s16 = srcv[pl.ds(eb + j * L, L)]
                d16 = dstv[pl.ds(eb + j * L, L)]
                dx = plsc.load_gather(px, [s16]) - plsc.load_gather(px, [d16])
                dy = plsc.load_gather(py, [s16]) - plsc.load_gather(py, [d16])
                dz = plsc.load_gather(pz, [s16]) - plsc.load_gather(pz, [d16])
                d2b[s, pl.ds(j * L, L)] = dx * dx + dy * dy + dz * dz

            @pl.loop(0, C)
            def _row(e):
                for j in range(Hw // L):
                    sl = pl.ds(j * L, L)
                    a = plsc.bitcast(g1[s, e, sl], jnp.bfloat16)
                    b = plsc.bitcast(g2[s, e, sl], jnp.bfloat16)
                    g1[s, e, sl] = plsc.bitcast(a + b, jnp.int32)

            issue(writeout_copies(c, s))

        issue(gather_copies(0, 0))
        npair = (NCH - 1) // 2

        @pl.loop(0, npair)
        def _pair(t):
            body(2 * t, 0)
            body(2 * t + 1, 1)

        body(NCH - 1, 0, tail=True)
        wait(writeout_copies(NCH - 1, 0))

    return k(P1, P2, src, dst, posx, posy, posz)


def _stage3_body(q_ref, d2_ref, w1c_ref, w2_ref, b2_ref, out_ref):
    qlo, qhi = _unpack_bf16_halves(q_ref[...])
    d2 = d2_ref[...]
    Hh = out_ref.shape[-1]
    w1c = w1c_ref[...]
    mlo = jnp.maximum(qlo + d2 * w1c[:, :Hh], 0.0).astype(jnp.bfloat16)
    mhi = jnp.maximum(qhi + d2 * w1c[:, Hh:], 0.0).astype(jnp.bfloat16)
    r = (jnp.dot(mlo, w2_ref[0], preferred_element_type=jnp.float32)
         + jnp.dot(mhi, w2_ref[1], preferred_element_type=jnp.float32)
         + b2_ref[...])
    r = jnp.maximum(r, 0.0)
    out_ref[0] = r[:, :Hh]
    out_ref[1] = r[:, Hh:]


def _edge_mlp(q, d2, w1c, W2s, b2):
    E, Hw = q.shape
    H = 2 * Hw
    BE = 1024
    return pl.pallas_call(
        _stage3_body,
        grid=(E // BE,),
        in_specs=[
            pl.BlockSpec((BE, Hw), lambda i: (i, 0)),
            pl.BlockSpec((BE, 1), lambda i: (i, 0)),
            pl.BlockSpec((1, H), lambda i: (0, 0)),
            pl.BlockSpec((2, Hw, H), lambda i: (0, 0, 0)),
            pl.BlockSpec((1, H), lambda i: (0, 0)),
        ],
        out_specs=pl.BlockSpec((2, BE, H // 2), lambda i: (0, i, 0)),
        out_shape=jax.ShapeDtypeStruct((2, E, H // 2), jnp.float32),
    )(q, d2, w1c, W2s, b2)


def _scatter_add(m2, dst3, N):
    _, E, Hh = m2.shape
    EPS = E // NS
    C = 80
    NCH = EPS // C
    RPT = (N // NS) // 8 * 8
    RPT_LAST = N - (NS - 1) * RPT

    mesh = plsc.VectorSubcoreMesh(core_axis_name="c", subcore_axis_name="s")

    @functools.partial(
        pl.kernel,
        out_type=jax.ShapeDtypeStruct((2, N, Hh), jnp.float32),
        mesh=mesh,
        scratch_types=[
            pltpu.VMEM((128, C), jnp.int32),
            pltpu.VMEM((2, C, Hh), jnp.float32),
            pltpu.VMEM((L, Hh), jnp.float32),
            pltpu.VMEM_SHARED((N, Hh), jnp.float32),
            [pltpu.SemaphoreType.DMA] * 2,
        ],
    )
    def k(m_hbm, dst_hbm, out_hbm, idxb, mbuf, zbuf, acc, seml):
        cid = lax.axis_index("c")
        sid = lax.axis_index("s")
        ebase = sid * EPS
        zero = jnp.zeros((L,), jnp.float32)

        @pl.loop(0, L)
        def _z(r):
            for j in range(Hh // L):
                zbuf[r, pl.ds(j * L, L)] = zero

        rbase = sid * RPT

        @pl.when(sid < NS - 1)
        def _():
            @pl.loop(0, RPT // L)
            def _zc(t):
                pltpu.sync_copy(zbuf, acc.at[pl.ds(rbase + t * L, L)])

        @pl.when(sid == NS - 1)
        def _():
            @pl.loop(0, RPT_LAST // L)
            def _zc(t):
                pltpu.sync_copy(zbuf, acc.at[pl.ds(rbase + t * L, L)])

        plsc.subcore_barrier()

        def load_copy(c, s):
            return pltpu.make_async_copy(
                m_hbm.at[cid, pl.ds(ebase + c * C, C)], mbuf.at[s], seml[s])

        def phase(cbase, nph):
            pltpu.sync_copy(dst_hbm.at[sid, pl.ds(cbase, nph)],
                            idxb.at[pl.ds(0, nph)])
            load_copy(cbase, 0).start()

            @pl.loop(0, nph // 2)
            def _chunk(t):
                for s in (0, 1):
                    k = 2 * t + s
                    c = cbase + k
                    load_copy(c, s).wait()

                    @pl.when(k + 1 < nph)
                    def _():
                        load_copy(c + 1, s ^ 1).start()

                    pltpu.sync_copy(mbuf.at[s], acc.at[idxb.at[k]], add=True)

        phase(0, 128)
        phase(128, NCH - 128)

        plsc.subcore_barrier()

        @pl.when(sid < NS - 1)
        def _():
            pltpu.sync_copy(acc.at[pl.ds(rbase, RPT)],
                            out_hbm.at[cid, pl.ds(rbase, RPT)])

        @pl.when(sid == NS - 1)
        def _():
            pltpu.sync_copy(acc.at[pl.ds(rbase, RPT_LAST)],
                            out_hbm.at[cid, pl.ds(rbase, RPT_LAST)])

    return k(m2, dst3)


def _stage5_body(magg_ref, h_ref, w3_ref, b3_ref, out_ref):
    r = (jnp.dot(magg_ref[0], w3_ref[0], preferred_element_type=jnp.float32)
         + jnp.dot(magg_ref[1], w3_ref[1], preferred_element_type=jnp.float32)
         + b3_ref[...])
    out_ref[...] = h_ref[...] + jnp.maximum(r, 0.0)


def _node_update(magg, h2, W3s, b3):
    N, D = h2.shape
    Hh = W3s.shape[1]
    BN = 1000
    return pl.pallas_call(
        _stage5_body,
        grid=(N // BN,),
        in_specs=[
            pl.BlockSpec((2, BN, Hh), lambda i: (0, i, 0)),
            pl.BlockSpec((BN, D), lambda i: (i, 0)),
            pl.BlockSpec((2, Hh, D), lambda i: (0, 0, 0)),
            pl.BlockSpec((1, D), lambda i: (0, 0)),
        ],
        out_specs=pl.BlockSpec((BN, D), lambda i: (i, 0)),
        out_shape=jax.ShapeDtypeStruct((N, D), jnp.float32),
    )(magg, h2, W3s, b3)


def kernel(h, pos, edge_index, W1, b1, W2, b2, W3, b3):
    B, N, D = h.shape
    E = edge_index.shape[1]
    H = W2.shape[0]

    h2 = h[0]
    posx, posy, posz = pos[0, :, 0], pos[0, :, 1], pos[0, :, 2]
    src = edge_index[0]
    dst = edge_index[1]
    W1a = W1[:D]
    W1b = W1[D:2 * D]
    w1c = W1[2 * D:2 * D + 1]

    P1, P2 = _node_proj(h2, W1a, W1b, b1.reshape(1, H))
    q, d2 = _edge_gather(P1, P2, src, dst, posx, posy, posz)
    m2 = _edge_mlp(q, d2.reshape(E, 1), w1c,
                   W2.astype(jnp.bfloat16).reshape(2, H // 2, H),
                   b2.reshape(1, H))
    dst3 = dst.reshape(NS, (E // NS) // 80, 80)
    magg = _scatter_add(m2, dst3, N)
    h_out = _node_update(magg, h2, W3.reshape(2, H // 2, D), b3.reshape(1, D))
    return (h_out[None], pos)
